# baseline (speedup 1.0000x reference)
import jax
import jax.numpy as jnp
from jax.experimental import pallas as pl
from jax.experimental.pallas import tpu as pltpu


def _pool_kernel(m_ref, x_ref, o_ref):
    m = m_ref[...]
    x = x_ref[...]
    acc = jnp.sum(x * m[:, :, None], axis=1)
    cnt = jnp.sum(m, axis=1, keepdims=True)
    o_ref[...] = (acc / (cnt + 1e-16)).astype(o_ref.dtype)


def _pick_tb(B, S, D, itemsize, budget=56 << 20):
    for tb in (32, 16, 8):
        if B % tb == 0 and 2 * tb * S * D * itemsize <= budget:
            return tb
    return 8 if B % 8 == 0 else B


def kernel(x, mask):
    B, S, D = x.shape
    TB = _pick_tb(B, S, D, x.dtype.itemsize)
    grid = (B // TB,)

    mask2d = mask.reshape(B, S)
    cost = pl.CostEstimate(
        flops=3 * B * S * D,
        transcendentals=0,
        bytes_accessed=(x.size * x.dtype.itemsize
                        + mask2d.size * mask2d.dtype.itemsize
                        + B * D * x.dtype.itemsize),
    )
    return pl.pallas_call(
        _pool_kernel,
        out_shape=jax.ShapeDtypeStruct((B, D), x.dtype),
        grid=grid,
        in_specs=[
            pl.BlockSpec((TB, S), lambda b: (b, 0)),
            pl.BlockSpec((TB, S, D), lambda b: (b, 0, 0)),
        ],
        out_specs=pl.BlockSpec((TB, D), lambda b: (b, 0)),
        compiler_params=pltpu.CompilerParams(
            dimension_semantics=("parallel",),
            vmem_limit_bytes=64 << 20,
        ),
        cost_estimate=cost,
    )(mask2d, x)


# speedup vs baseline: 1.0091x; 1.0091x over previous
import jax
import jax.numpy as jnp
from jax.experimental import pallas as pl
from jax.experimental.pallas import tpu as pltpu


def _pool_kernel(m_ref, x_ref, o_ref):
    m = m_ref[...]
    x = x_ref[...]
    acc = jnp.sum(x * m[:, :, None], axis=1)
    cnt = jnp.sum(m, axis=1, keepdims=True)
    o_ref[...] = (acc / (cnt + 1e-16)).astype(o_ref.dtype)


def _pick_tb(B, S, D, itemsize, budget=28 << 20):
    for tb in (32, 16, 8):
        if B % tb == 0 and 2 * tb * S * D * itemsize <= budget:
            return tb
    return 8 if B % 8 == 0 else B


def kernel(x, mask):
    B, S, D = x.shape
    TB = _pick_tb(B, S, D, x.dtype.itemsize)
    grid = (B // TB,)

    mask2d = mask.reshape(B, S)
    cost = pl.CostEstimate(
        flops=3 * B * S * D,
        transcendentals=0,
        bytes_accessed=(x.size * x.dtype.itemsize
                        + mask2d.size * mask2d.dtype.itemsize
                        + B * D * x.dtype.itemsize),
    )
    return pl.pallas_call(
        _pool_kernel,
        out_shape=jax.ShapeDtypeStruct((B, D), x.dtype),
        grid=grid,
        in_specs=[
            pl.BlockSpec((TB, S), lambda b: (b, 0)),
            pl.BlockSpec((TB, S, D), lambda b: (b, 0, 0)),
        ],
        out_specs=pl.BlockSpec((TB, D), lambda b: (b, 0)),
        compiler_params=pltpu.CompilerParams(
            dimension_semantics=("parallel",),
            vmem_limit_bytes=64 << 20,
        ),
        cost_estimate=cost,
    )(mask2d, x)


# speedup vs baseline: 1.0094x; 1.0003x over previous
import jax
import jax.numpy as jnp
from jax.experimental import pallas as pl
from jax.experimental.pallas import tpu as pltpu


def _pool_kernel2(m_ref, xa_ref, xb_ref, o_ref, *, hs):
    m = m_ref[...]
    ma = m[:, :hs]
    mb = m[:, hs:]
    acc = (jnp.sum(xa_ref[...] * ma[:, :, None], axis=1)
           + jnp.sum(xb_ref[...] * mb[:, :, None], axis=1))
    cnt = jnp.sum(m, axis=1, keepdims=True)
    o_ref[...] = (acc / (cnt + 1e-16)).astype(o_ref.dtype)


def kernel(x, mask):
    import functools
    B, S, D = x.shape
    TB = 8
    hs = S // 2
    grid = (B // TB,)
    mask2d = mask.reshape(B, S)
    return pl.pallas_call(
        functools.partial(_pool_kernel2, hs=hs),
        out_shape=jax.ShapeDtypeStruct((B, D), x.dtype),
        grid=grid,
        in_specs=[
            pl.BlockSpec((TB, S), lambda b: (b, 0)),
            pl.BlockSpec((TB, hs, D), lambda b: (b, 0, 0)),
            pl.BlockSpec((TB, hs, D), lambda b: (b, 1, 0)),
        ],
        out_specs=pl.BlockSpec((TB, D), lambda b: (b, 0)),
        compiler_params=pltpu.CompilerParams(
            dimension_semantics=("parallel",),
            vmem_limit_bytes=64 << 20,
        ),
    )(mask2d, x, x)


# speedup vs baseline: 1.0100x; 1.0006x over previous
import jax
import jax.numpy as jnp
from jax.experimental import pallas as pl
from jax.experimental.pallas import tpu as pltpu


def _pool_kernel(m_ref, x_ref, o_ref):
    m = m_ref[...]
    x = x_ref[...]
    acc = jnp.sum(x * m[:, :, None], axis=1)
    cnt = jnp.sum(m, axis=1, keepdims=True)
    o_ref[...] = (acc / (cnt + 1e-16)).astype(o_ref.dtype)


def _pick_tb(B, S, D, itemsize, budget=28 << 20):
    for tb in (32, 16, 8):
        if B % tb == 0 and 2 * tb * S * D * itemsize <= budget:
            return tb
    return 8 if B % 8 == 0 else B


def kernel(x, mask):
    B, S, D = x.shape
    TB = _pick_tb(B, S, D, x.dtype.itemsize)
    grid = (B // TB,)

    mask2d = mask.reshape(B, S)
    cost = pl.CostEstimate(
        flops=3 * B * S * D,
        transcendentals=0,
        bytes_accessed=(x.size * x.dtype.itemsize
                        + mask2d.size * mask2d.dtype.itemsize
                        + B * D * x.dtype.itemsize),
    )
    return pl.pallas_call(
        _pool_kernel,
        out_shape=jax.ShapeDtypeStruct((B, D), x.dtype),
        grid=grid,
        in_specs=[
            pl.BlockSpec((TB, S), lambda b: (b, 0)),
            pl.BlockSpec((TB, S, D), lambda b: (b, 0, 0)),
        ],
        out_specs=pl.BlockSpec((TB, D), lambda b: (b, 0)),
        compiler_params=pltpu.CompilerParams(
            dimension_semantics=("parallel",),
            vmem_limit_bytes=64 << 20,
        ),
        cost_estimate=cost,
    )(mask2d, x)
